# baseline (speedup 1.0000x reference)
import functools

import jax
import jax.numpy as jnp
from jax import lax
from jax.experimental import pallas as pl
from jax.experimental.pallas import tpu as pltpu
from jax.experimental.pallas import tpu_sc as plsc


def kernel(input, table):
    B, S, D = input.shape
    V = table.shape[0]
    F = S * D

    info = plsc.get_sparse_core_info()
    NC, NS = info.num_cores, info.num_subcores
    NW = NC * NS
    BPW = B // NW
    R = 8
    NDMA = BPW // R

    tbl1 = jnp.reshape(table, (V * D,))
    mesh = plsc.VectorSubcoreMesh(core_axis_name="c", subcore_axis_name="s")

    @functools.partial(
        pl.kernel,
        out_type=jax.ShapeDtypeStruct((B, F), jnp.float32),
        mesh=mesh,
        scratch_types=[
            pltpu.VMEM((R, F), jnp.float32),
            pltpu.SemaphoreType.DMA,
            pltpu.SemaphoreType.DMA,
        ],
    )
    def sc_broadcast(tbl_hbm, out_hbm, buf, fill_sem, out_sem):
        wid = lax.axis_index("s") * NC + lax.axis_index("c")
        base = wid * BPW
        for r in range(R):
            pltpu.async_copy(tbl_hbm.at[pl.ds(0, F)], buf.at[r], fill_sem)
        for r in range(R):
            pltpu.make_async_copy(
                tbl_hbm.at[pl.ds(0, F)], buf.at[r], fill_sem).wait()
        for i in range(NDMA):
            pltpu.async_copy(
                buf, out_hbm.at[pl.ds(base + i * R, R)], out_sem)
        for i in range(NDMA):
            pltpu.make_async_copy(
                buf, out_hbm.at[pl.ds(base + i * R, R)], out_sem).wait()

    out2 = sc_broadcast(tbl1)
    return jnp.reshape(out2, (B, S, D))
